# Initial kernel scaffold; baseline (speedup 1.0000x reference)
import jax
import jax.numpy as jnp
from jax.experimental import pallas as pl


def kernel(x, edge_index, W1, b1, W2, b2):
    raise NotImplementedError("write your pallas kernel here")


# speedup vs baseline: 16.6879x; 16.6879x over previous
import functools

import jax
import jax.numpy as jnp
from jax import lax
from jax.experimental import pallas as pl
from jax.experimental.pallas import tpu as pltpu
from jax.experimental.pallas import tpu_sc as plsc

N_NODES = 10000
FEAT = 128
NC = 2
NS = 16
NW = NC * NS
CH = 128

_MESH = plsc.VectorSubcoreMesh(
    core_axis_name="c", subcore_axis_name="s", num_cores=NC, num_subcores=NS)


def _zero_rows(buf, nrows):
  zero16 = jnp.zeros((16,), jnp.float32)

  def row(i, _):
    def col(j, _):
      buf[i, pl.ds(j * 16, 16)] = zero16
      return 0
    return lax.fori_loop(0, FEAT // 16, col, 0)

  lax.fori_loop(0, nrows, row, 0)


def _sc_hist(dst):
  e = dst.shape[0]
  ept = e // NW
  n_full = ept // CH
  tail = ept - n_full * CH
  zch = 624

  @functools.partial(
      pl.kernel,
      out_type=jax.ShapeDtypeStruct((NC * N_NODES,), jnp.float32),
      mesh=_MESH,
      scratch_types=[
          pltpu.VMEM((CH,), jnp.int32),
          pltpu.VMEM((tail,), jnp.int32),
          pltpu.VMEM((CH,), jnp.float32),
          pltpu.VMEM((zch,), jnp.float32),
          pltpu.VMEM_SHARED((N_NODES,), jnp.float32),
      ],
  )
  def hist_kernel(dst_hbm, out_hbm, idx_v, tidx_v, ones_v, zbuf, hist_s):
    cid = lax.axis_index("c")
    sid = lax.axis_index("s")
    wid = sid * NC + cid

    one16 = jnp.ones((16,), jnp.float32)
    zero16 = jnp.zeros((16,), jnp.float32)
    for j in range(CH // 16):
      ones_v[pl.ds(j * 16, 16)] = one16
    def zb(j, _):
      zbuf[pl.ds(j * 16, 16)] = zero16
      return 0
    lax.fori_loop(0, zch // 16, zb, 0)

    pltpu.sync_copy(zbuf, hist_s.at[pl.ds(sid * zch, zch)])
    @pl.when(sid == NS - 1)
    def _():
      pltpu.sync_copy(zbuf.at[pl.ds(0, 16)],
                      hist_s.at[pl.ds(NS * zch, N_NODES - NS * zch)])
    plsc.subcore_barrier()

    def body(ci, _):
      base = pl.multiple_of(wid * ept + ci * CH, 8)
      pltpu.sync_copy(dst_hbm.at[pl.ds(base, CH)], idx_v)
      pltpu.sync_copy(ones_v, hist_s.at[idx_v], add=True)
      return 0
    lax.fori_loop(0, n_full, body, 0)
    if tail:
      base = pl.multiple_of(wid * ept + n_full * CH, 8)
      pltpu.sync_copy(dst_hbm.at[pl.ds(base, tail)], tidx_v)
      pltpu.sync_copy(ones_v.at[pl.ds(0, tail)], hist_s.at[tidx_v], add=True)

    plsc.subcore_barrier()
    obase = pl.multiple_of(cid * N_NODES + sid * zch, 8)
    pltpu.sync_copy(hist_s.at[pl.ds(sid * zch, zch)], zbuf)
    pltpu.sync_copy(zbuf, out_hbm.at[pl.ds(obase, zch)])
    @pl.when(sid == NS - 1)
    def _():
      tb = pl.multiple_of(cid * N_NODES + NS * zch, 8)
      pltpu.sync_copy(hist_s.at[pl.ds(NS * zch, N_NODES - NS * zch)],
                      ones_v.at[pl.ds(0, N_NODES - NS * zch)])
      pltpu.sync_copy(ones_v.at[pl.ds(0, N_NODES - NS * zch)],
                      out_hbm.at[pl.ds(tb, N_NODES - NS * zch)])

  return hist_kernel(dst)


def _sc_mp(g, src, dst):
  e = src.shape[0]
  ept = e // NW
  n_full = ept // CH
  tail = ept - n_full * CH
  rpt = 624

  @functools.partial(
      pl.kernel,
      out_type=jax.ShapeDtypeStruct((NC, N_NODES, FEAT), jnp.float32),
      mesh=_MESH,
      scratch_types=[
          pltpu.VMEM((CH,), jnp.int32),
          pltpu.VMEM((CH,), jnp.int32),
          pltpu.VMEM((CH, FEAT), jnp.float32),
          pltpu.VMEM((tail,), jnp.int32),
          pltpu.VMEM((tail,), jnp.int32),
          pltpu.VMEM((tail, FEAT), jnp.float32),
          pltpu.VMEM_SHARED((N_NODES, FEAT), jnp.float32),
          pltpu.SemaphoreType.DMA,
      ],
  )
  def mp_kernel(g_hbm, src_hbm, dst_hbm, out_hbm,
                src_v, dst_v, rows_v, tsrc_v, tdst_v, trows_v, acc_s, sem):
    cid = lax.axis_index("c")
    sid = lax.axis_index("s")
    wid = sid * NC + cid

    _zero_rows(rows_v, CH)
    r0 = pl.multiple_of(sid * rpt, 8)
    for k in range(rpt // CH):
      pltpu.sync_copy(rows_v, acc_s.at[pl.ds(r0 + k * CH, CH)])
    rem = rpt - (rpt // CH) * CH
    if rem:
      pltpu.sync_copy(rows_v.at[pl.ds(0, rem)],
                      acc_s.at[pl.ds(r0 + (rpt // CH) * CH, rem)])
    ntail = N_NODES - NS * rpt
    @pl.when(sid == NS - 1)
    def _():
      pltpu.sync_copy(rows_v.at[pl.ds(0, ntail)],
                      acc_s.at[pl.ds(NS * rpt, ntail)])
    plsc.subcore_barrier()

    def body(ci, _):
      base = pl.multiple_of(wid * ept + ci * CH, 8)
      pltpu.sync_copy(src_hbm.at[pl.ds(base, CH)], src_v)
      pltpu.sync_copy(dst_hbm.at[pl.ds(base, CH)], dst_v)
      pltpu.async_copy(g_hbm.at[src_v], rows_v, sem).wait()
      pltpu.sync_copy(rows_v, acc_s.at[dst_v], add=True)
      return 0
    lax.fori_loop(0, n_full, body, 0)
    if tail:
      base = pl.multiple_of(wid * ept + n_full * CH, 8)
      pltpu.sync_copy(src_hbm.at[pl.ds(base, tail)], tsrc_v)
      pltpu.sync_copy(dst_hbm.at[pl.ds(base, tail)], tdst_v)
      pltpu.async_copy(g_hbm.at[tsrc_v], trows_v, sem).wait()
      pltpu.sync_copy(trows_v, acc_s.at[tdst_v], add=True)

    plsc.subcore_barrier()
    for k in range(rpt // CH):
      pltpu.sync_copy(acc_s.at[pl.ds(r0 + k * CH, CH)], rows_v)
      pltpu.sync_copy(rows_v, out_hbm.at[cid, pl.ds(r0 + k * CH, CH)])
    if rem:
      rb = pl.multiple_of(r0 + (rpt // CH) * CH, 8)
      pltpu.sync_copy(acc_s.at[pl.ds(rb, rem)], rows_v.at[pl.ds(0, rem)])
      pltpu.sync_copy(rows_v.at[pl.ds(0, rem)],
                      out_hbm.at[cid, pl.ds(rb, rem)])
    @pl.when(sid == NS - 1)
    def _():
      pltpu.sync_copy(acc_s.at[pl.ds(NS * rpt, ntail)],
                      rows_v.at[pl.ds(0, ntail)])
      pltpu.sync_copy(rows_v.at[pl.ds(0, ntail)],
                      out_hbm.at[cid, pl.ds(NS * rpt, ntail)])

  return mp_kernel(g, src, dst)


_R = 1000


def _tc_scale_matmul(hist_t, x, W1):
  def body(hist_ref, x_ref, w_ref, g_ref, dinv_ref):
    deg = hist_ref[:, 0:1] + hist_ref[:, 1:2] + 1.0
    dinv = lax.rsqrt(deg)
    dinv_ref[...] = dinv
    g_ref[...] = dinv * jnp.dot(x_ref[...], w_ref[...],
                                preferred_element_type=jnp.float32)

  return pl.pallas_call(
      body,
      grid=(N_NODES // _R,),
      in_specs=[
          pl.BlockSpec((_R, NC), lambda i: (i, 0)),
          pl.BlockSpec((_R, FEAT), lambda i: (i, 0)),
          pl.BlockSpec((FEAT, FEAT), lambda i: (0, 0)),
      ],
      out_specs=[
          pl.BlockSpec((_R, FEAT), lambda i: (i, 0)),
          pl.BlockSpec((_R, 1), lambda i: (i, 0)),
      ],
      out_shape=[
          jax.ShapeDtypeStruct((N_NODES, FEAT), jnp.float32),
          jax.ShapeDtypeStruct((N_NODES, 1), jnp.float32),
      ],
  )(hist_t, x, W1)


def _tc_layer1_matmul2(acc, g1s, dinv, b1, W2):
  def body(acc_ref, g_ref, dinv_ref, b_ref, w_ref, o_ref):
    s = acc_ref[0] + acc_ref[1] + g_ref[...]
    h = jnp.maximum(dinv_ref[...] * s + b_ref[...], 0.0)
    o_ref[...] = dinv_ref[...] * jnp.dot(h, w_ref[...],
                                         preferred_element_type=jnp.float32)

  return pl.pallas_call(
      body,
      grid=(N_NODES // _R,),
      in_specs=[
          pl.BlockSpec((NC, _R, FEAT), lambda i: (0, i, 0)),
          pl.BlockSpec((_R, FEAT), lambda i: (i, 0)),
          pl.BlockSpec((_R, 1), lambda i: (i, 0)),
          pl.BlockSpec((1, FEAT), lambda i: (0, 0)),
          pl.BlockSpec((FEAT, FEAT), lambda i: (0, 0)),
      ],
      out_specs=pl.BlockSpec((_R, FEAT), lambda i: (i, 0)),
      out_shape=jax.ShapeDtypeStruct((N_NODES, FEAT), jnp.float32),
  )(acc, g1s, dinv, b1, W2)


def _tc_layer2_logsoftmax(acc, g2s, dinv, b2):
  def body(acc_ref, g_ref, dinv_ref, b_ref, o_ref):
    z = dinv_ref[...] * (acc_ref[0] + acc_ref[1] + g_ref[...]) + b_ref[...]
    m = jnp.max(z, axis=-1, keepdims=True)
    ez = jnp.exp(z - m)
    o_ref[...] = z - m - jnp.log(jnp.sum(ez, axis=-1, keepdims=True))

  return pl.pallas_call(
      body,
      grid=(N_NODES // _R,),
      in_specs=[
          pl.BlockSpec((NC, _R, FEAT), lambda i: (0, i, 0)),
          pl.BlockSpec((_R, FEAT), lambda i: (i, 0)),
          pl.BlockSpec((_R, 1), lambda i: (i, 0)),
          pl.BlockSpec((1, FEAT), lambda i: (0, 0)),
      ],
      out_specs=pl.BlockSpec((_R, FEAT), lambda i: (i, 0)),
      out_shape=jax.ShapeDtypeStruct((N_NODES, FEAT), jnp.float32),
  )(acc, g2s, dinv, b2)


def kernel(x, edge_index, W1, b1, W2, b2):
  ei = edge_index.astype(jnp.int32)
  src, dst = ei[0], ei[1]

  hist = _sc_hist(dst).reshape(NC, N_NODES)
  hist_t = hist.T
  g1s, dinv = _tc_scale_matmul(hist_t, x, W1)
  acc1 = _sc_mp(g1s, src, dst)
  g2s = _tc_layer1_matmul2(acc1, g1s, dinv, b1.reshape(1, FEAT), W2)
  acc2 = _sc_mp(g2s, src, dst)
  return _tc_layer2_logsoftmax(acc2, g2s, dinv, b2.reshape(1, FEAT))


# speedup vs baseline: 22.6660x; 1.3582x over previous
import functools

import jax
import jax.numpy as jnp
from jax import lax
from jax.experimental import pallas as pl
from jax.experimental.pallas import tpu as pltpu
from jax.experimental.pallas import tpu_sc as plsc

N_NODES = 10000
FEAT = 128
NC = 2
NS = 16
NW = NC * NS
CH = 128

_MESH = plsc.VectorSubcoreMesh(
    core_axis_name="c", subcore_axis_name="s", num_cores=NC, num_subcores=NS)


def _zero_rows(buf, nrows):
  zero16 = jnp.zeros((16,), jnp.float32)

  def row(i, _):
    def col(j, _):
      buf[i, pl.ds(j * 16, 16)] = zero16
      return 0
    return lax.fori_loop(0, FEAT // 16, col, 0)

  lax.fori_loop(0, nrows, row, 0)


def _sc_hist(dst):
  e = dst.shape[0]
  ept = e // NW
  n_full = ept // CH
  tail = ept - n_full * CH
  zch = 624

  @functools.partial(
      pl.kernel,
      out_type=jax.ShapeDtypeStruct((NC * N_NODES,), jnp.float32),
      mesh=_MESH,
      scratch_types=[
          pltpu.VMEM((CH,), jnp.int32),
          pltpu.VMEM((tail,), jnp.int32),
          pltpu.VMEM((CH,), jnp.float32),
          pltpu.VMEM((zch,), jnp.float32),
          pltpu.VMEM_SHARED((N_NODES,), jnp.float32),
      ],
  )
  def hist_kernel(dst_hbm, out_hbm, idx_v, tidx_v, ones_v, zbuf, hist_s):
    cid = lax.axis_index("c")
    sid = lax.axis_index("s")
    wid = sid * NC + cid

    one16 = jnp.ones((16,), jnp.float32)
    zero16 = jnp.zeros((16,), jnp.float32)
    for j in range(CH // 16):
      ones_v[pl.ds(j * 16, 16)] = one16
    def zb(j, _):
      zbuf[pl.ds(j * 16, 16)] = zero16
      return 0
    lax.fori_loop(0, zch // 16, zb, 0)

    pltpu.sync_copy(zbuf, hist_s.at[pl.ds(sid * zch, zch)])
    @pl.when(sid == NS - 1)
    def _():
      pltpu.sync_copy(zbuf.at[pl.ds(0, 16)],
                      hist_s.at[pl.ds(NS * zch, N_NODES - NS * zch)])
    plsc.subcore_barrier()

    def body(ci, _):
      base = pl.multiple_of(wid * ept + ci * CH, 8)
      pltpu.sync_copy(dst_hbm.at[pl.ds(base, CH)], idx_v)
      pltpu.sync_copy(ones_v, hist_s.at[idx_v], add=True)
      return 0
    lax.fori_loop(0, n_full, body, 0)
    if tail:
      base = pl.multiple_of(wid * ept + n_full * CH, 8)
      pltpu.sync_copy(dst_hbm.at[pl.ds(base, tail)], tidx_v)
      pltpu.sync_copy(ones_v.at[pl.ds(0, tail)], hist_s.at[tidx_v], add=True)

    plsc.subcore_barrier()
    obase = pl.multiple_of(cid * N_NODES + sid * zch, 8)
    pltpu.sync_copy(hist_s.at[pl.ds(sid * zch, zch)], zbuf)
    pltpu.sync_copy(zbuf, out_hbm.at[pl.ds(obase, zch)])
    @pl.when(sid == NS - 1)
    def _():
      tb = pl.multiple_of(cid * N_NODES + NS * zch, 8)
      pltpu.sync_copy(hist_s.at[pl.ds(NS * zch, N_NODES - NS * zch)],
                      ones_v.at[pl.ds(0, N_NODES - NS * zch)])
      pltpu.sync_copy(ones_v.at[pl.ds(0, N_NODES - NS * zch)],
                      out_hbm.at[pl.ds(tb, N_NODES - NS * zch)])

  return hist_kernel(dst)


def _sc_mp(g, src, dst):
  e = src.shape[0]
  ept = e // NW
  n_full = ept // CH
  tail = ept - n_full * CH
  rpt = 624

  assert n_full % 2 == 0 and n_full >= 4
  half = n_full // 2

  @functools.partial(
      pl.kernel,
      out_type=jax.ShapeDtypeStruct((NC, N_NODES, FEAT), jnp.float32),
      mesh=_MESH,
      scratch_types=[
          pltpu.VMEM((CH,), jnp.int32),
          pltpu.VMEM((CH,), jnp.int32),
          pltpu.VMEM((CH,), jnp.int32),
          pltpu.VMEM((CH,), jnp.int32),
          pltpu.VMEM((CH, FEAT), jnp.float32),
          pltpu.VMEM((CH, FEAT), jnp.float32),
          pltpu.VMEM((tail,), jnp.int32),
          pltpu.VMEM((tail,), jnp.int32),
          pltpu.VMEM((tail, FEAT), jnp.float32),
          pltpu.VMEM_SHARED((N_NODES, FEAT), jnp.float32),
          pltpu.SemaphoreType.DMA,
          pltpu.SemaphoreType.DMA,
          pltpu.SemaphoreType.DMA,
          pltpu.SemaphoreType.DMA,
          pltpu.SemaphoreType.DMA,
          pltpu.SemaphoreType.DMA,
      ],
  )
  def mp_kernel(g_hbm, src_hbm, dst_hbm, out_hbm,
                srcv0, dstv0, srcv1, dstv1, rows0, rows1,
                tsrc_v, tdst_v, trows_v, acc_s,
                gsem0, gsem1, ssem0, ssem1, isem0, isem1):
    cid = lax.axis_index("c")
    sid = lax.axis_index("s")
    wid = sid * NC + cid

    _zero_rows(rows0, CH)
    r0 = pl.multiple_of(sid * rpt, 8)
    for k in range(rpt // CH):
      pltpu.sync_copy(rows0, acc_s.at[pl.ds(r0 + k * CH, CH)])
    rem = rpt - (rpt // CH) * CH
    if rem:
      pltpu.sync_copy(rows0.at[pl.ds(0, rem)],
                      acc_s.at[pl.ds(r0 + (rpt // CH) * CH, rem)])
    ntail = N_NODES - NS * rpt
    @pl.when(sid == NS - 1)
    def _():
      pltpu.sync_copy(rows0.at[pl.ds(0, ntail)],
                      acc_s.at[pl.ds(NS * rpt, ntail)])
    plsc.subcore_barrier()

    ebase = wid * ept

    def idx_start(c, sv, dv, isem):
      b = pl.multiple_of(ebase + c * CH, 8)
      pltpu.make_async_copy(src_hbm.at[pl.ds(b, CH)], sv, isem).start()
      pltpu.make_async_copy(dst_hbm.at[pl.ds(b, CH)], dv, isem).start()

    def idx_wait(c, sv, dv, isem):
      b = pl.multiple_of(ebase + c * CH, 8)
      pltpu.make_async_copy(src_hbm.at[pl.ds(b, CH)], sv, isem).wait()
      pltpu.make_async_copy(dst_hbm.at[pl.ds(b, CH)], dv, isem).wait()

    idx_start(0, srcv0, dstv0, isem0)
    idx_wait(0, srcv0, dstv0, isem0)
    pltpu.make_async_copy(g_hbm.at[srcv0], rows0, gsem0).start()

    def body(o, _):
      pltpu.make_async_copy(g_hbm.at[srcv0], rows0, gsem0).wait()
      pltpu.make_async_copy(rows0, acc_s.at[dstv0], ssem0).start(add=True)
      @pl.when(o > 0)
      def _():
        pltpu.make_async_copy(rows1, acc_s.at[dstv1], ssem1).wait()
      c1 = 2 * o + 1
      idx_start(c1, srcv1, dstv1, isem1)
      idx_wait(c1, srcv1, dstv1, isem1)
      pltpu.make_async_copy(g_hbm.at[srcv1], rows1, gsem1).start()
      pltpu.make_async_copy(g_hbm.at[srcv1], rows1, gsem1).wait()
      pltpu.make_async_copy(rows1, acc_s.at[dstv1], ssem1).start(add=True)
      pltpu.make_async_copy(rows0, acc_s.at[dstv0], ssem0).wait()
      @pl.when(o < half - 1)
      def _():
        c2 = 2 * o + 2
        idx_start(c2, srcv0, dstv0, isem0)
        idx_wait(c2, srcv0, dstv0, isem0)
        pltpu.make_async_copy(g_hbm.at[srcv0], rows0, gsem0).start()
      return 0
    lax.fori_loop(0, half, body, 0)
    pltpu.make_async_copy(rows1, acc_s.at[dstv1], ssem1).wait()

    if tail:
      b = pl.multiple_of(ebase + n_full * CH, 8)
      pltpu.sync_copy(src_hbm.at[pl.ds(b, tail)], tsrc_v)
      pltpu.sync_copy(dst_hbm.at[pl.ds(b, tail)], tdst_v)
      pltpu.async_copy(g_hbm.at[tsrc_v], trows_v, gsem0).wait()
      pltpu.sync_copy(trows_v, acc_s.at[tdst_v], add=True)

    plsc.subcore_barrier()
    for k in range(rpt // CH):
      pltpu.sync_copy(acc_s.at[pl.ds(r0 + k * CH, CH)], rows0)
      pltpu.sync_copy(rows0, out_hbm.at[cid, pl.ds(r0 + k * CH, CH)])
    if rem:
      rb = pl.multiple_of(r0 + (rpt // CH) * CH, 8)
      pltpu.sync_copy(acc_s.at[pl.ds(rb, rem)], rows0.at[pl.ds(0, rem)])
      pltpu.sync_copy(rows0.at[pl.ds(0, rem)],
                      out_hbm.at[cid, pl.ds(rb, rem)])
    @pl.when(sid == NS - 1)
    def _():
      pltpu.sync_copy(acc_s.at[pl.ds(NS * rpt, ntail)],
                      rows0.at[pl.ds(0, ntail)])
      pltpu.sync_copy(rows0.at[pl.ds(0, ntail)],
                      out_hbm.at[cid, pl.ds(NS * rpt, ntail)])

  return mp_kernel(g, src, dst)


_R = 1000


def _tc_scale_matmul(hist_t, x, W1):
  def body(hist_ref, x_ref, w_ref, g_ref, dinv_ref):
    deg = hist_ref[:, 0:1] + hist_ref[:, 1:2] + 1.0
    dinv = lax.rsqrt(deg)
    dinv_ref[...] = dinv
    g_ref[...] = dinv * jnp.dot(x_ref[...], w_ref[...],
                                preferred_element_type=jnp.float32)

  return pl.pallas_call(
      body,
      grid=(N_NODES // _R,),
      in_specs=[
          pl.BlockSpec((_R, NC), lambda i: (i, 0)),
          pl.BlockSpec((_R, FEAT), lambda i: (i, 0)),
          pl.BlockSpec((FEAT, FEAT), lambda i: (0, 0)),
      ],
      out_specs=[
          pl.BlockSpec((_R, FEAT), lambda i: (i, 0)),
          pl.BlockSpec((_R, 1), lambda i: (i, 0)),
      ],
      out_shape=[
          jax.ShapeDtypeStruct((N_NODES, FEAT), jnp.float32),
          jax.ShapeDtypeStruct((N_NODES, 1), jnp.float32),
      ],
  )(hist_t, x, W1)


def _tc_layer1_matmul2(acc, g1s, dinv, b1, W2):
  def body(acc_ref, g_ref, dinv_ref, b_ref, w_ref, o_ref):
    s = acc_ref[0] + acc_ref[1] + g_ref[...]
    h = jnp.maximum(dinv_ref[...] * s + b_ref[...], 0.0)
    o_ref[...] = dinv_ref[...] * jnp.dot(h, w_ref[...],
                                         preferred_element_type=jnp.float32)

  return pl.pallas_call(
      body,
      grid=(N_NODES // _R,),
      in_specs=[
          pl.BlockSpec((NC, _R, FEAT), lambda i: (0, i, 0)),
          pl.BlockSpec((_R, FEAT), lambda i: (i, 0)),
          pl.BlockSpec((_R, 1), lambda i: (i, 0)),
          pl.BlockSpec((1, FEAT), lambda i: (0, 0)),
          pl.BlockSpec((FEAT, FEAT), lambda i: (0, 0)),
      ],
      out_specs=pl.BlockSpec((_R, FEAT), lambda i: (i, 0)),
      out_shape=jax.ShapeDtypeStruct((N_NODES, FEAT), jnp.float32),
  )(acc, g1s, dinv, b1, W2)


def _tc_layer2_logsoftmax(acc, g2s, dinv, b2):
  def body(acc_ref, g_ref, dinv_ref, b_ref, o_ref):
    z = dinv_ref[...] * (acc_ref[0] + acc_ref[1] + g_ref[...]) + b_ref[...]
    m = jnp.max(z, axis=-1, keepdims=True)
    ez = jnp.exp(z - m)
    o_ref[...] = z - m - jnp.log(jnp.sum(ez, axis=-1, keepdims=True))

  return pl.pallas_call(
      body,
      grid=(N_NODES // _R,),
      in_specs=[
          pl.BlockSpec((NC, _R, FEAT), lambda i: (0, i, 0)),
          pl.BlockSpec((_R, FEAT), lambda i: (i, 0)),
          pl.BlockSpec((_R, 1), lambda i: (i, 0)),
          pl.BlockSpec((1, FEAT), lambda i: (0, 0)),
      ],
      out_specs=pl.BlockSpec((_R, FEAT), lambda i: (i, 0)),
      out_shape=jax.ShapeDtypeStruct((N_NODES, FEAT), jnp.float32),
  )(acc, g2s, dinv, b2)


def kernel(x, edge_index, W1, b1, W2, b2):
  ei = edge_index.astype(jnp.int32)
  src, dst = ei[0], ei[1]

  hist = _sc_hist(dst).reshape(NC, N_NODES)
  hist_t = hist.T
  g1s, dinv = _tc_scale_matmul(hist_t, x, W1)
  acc1 = _sc_mp(g1s, src, dst)
  g2s = _tc_layer1_matmul2(acc1, g1s, dinv, b1.reshape(1, FEAT), W2)
  acc2 = _sc_mp(g2s, src, dst)
  return _tc_layer2_logsoftmax(acc2, g2s, dinv, b2.reshape(1, FEAT))


# speedup vs baseline: 29.0524x; 1.2818x over previous
import functools

import jax
import jax.numpy as jnp
from jax import lax
from jax.experimental import pallas as pl
from jax.experimental.pallas import tpu as pltpu
from jax.experimental.pallas import tpu_sc as plsc

N_NODES = 10000
FEAT = 128
HF = FEAT // 2
NC = 2
NS = 16
NW = NC * NS
CH = 128

_MESH = plsc.VectorSubcoreMesh(
    core_axis_name="c", subcore_axis_name="s", num_cores=NC, num_subcores=NS)


def _zero_rows(buf, nrows, width):
  zero16 = jnp.zeros((16,), jnp.float32)

  def row(i, _):
    def col(j, _):
      buf[i, pl.ds(j * 16, 16)] = zero16
      return 0
    return lax.fori_loop(0, width // 16, col, 0)

  lax.fori_loop(0, nrows, row, 0)


def _sc_hist(dst):
  e = dst.shape[0]
  ept = e // NW
  n_full = ept // CH
  tail = ept - n_full * CH
  zch = 624

  @functools.partial(
      pl.kernel,
      out_type=jax.ShapeDtypeStruct((NC * N_NODES,), jnp.float32),
      mesh=_MESH,
      scratch_types=[
          pltpu.VMEM((CH,), jnp.int32),
          pltpu.VMEM((tail,), jnp.int32),
          pltpu.VMEM((CH,), jnp.float32),
          pltpu.VMEM((zch,), jnp.float32),
          pltpu.VMEM_SHARED((N_NODES,), jnp.float32),
      ],
  )
  def hist_kernel(dst_hbm, out_hbm, idx_v, tidx_v, ones_v, zbuf, hist_s):
    cid = lax.axis_index("c")
    sid = lax.axis_index("s")
    wid = sid * NC + cid

    one16 = jnp.ones((16,), jnp.float32)
    zero16 = jnp.zeros((16,), jnp.float32)
    for j in range(CH // 16):
      ones_v[pl.ds(j * 16, 16)] = one16
    def zb(j, _):
      zbuf[pl.ds(j * 16, 16)] = zero16
      return 0
    lax.fori_loop(0, zch // 16, zb, 0)

    pltpu.sync_copy(zbuf, hist_s.at[pl.ds(sid * zch, zch)])
    @pl.when(sid == NS - 1)
    def _():
      pltpu.sync_copy(zbuf.at[pl.ds(0, 16)],
                      hist_s.at[pl.ds(NS * zch, N_NODES - NS * zch)])
    plsc.subcore_barrier()

    def body(ci, _):
      base = pl.multiple_of(wid * ept + ci * CH, 8)
      pltpu.sync_copy(dst_hbm.at[pl.ds(base, CH)], idx_v)
      pltpu.sync_copy(ones_v, hist_s.at[idx_v], add=True)
      return 0
    lax.fori_loop(0, n_full, body, 0)
    if tail:
      base = pl.multiple_of(wid * ept + n_full * CH, 8)
      pltpu.sync_copy(dst_hbm.at[pl.ds(base, tail)], tidx_v)
      pltpu.sync_copy(ones_v.at[pl.ds(0, tail)], hist_s.at[tidx_v], add=True)

    plsc.subcore_barrier()
    obase = pl.multiple_of(cid * N_NODES + sid * zch, 8)
    pltpu.sync_copy(hist_s.at[pl.ds(sid * zch, zch)], zbuf)
    pltpu.sync_copy(zbuf, out_hbm.at[pl.ds(obase, zch)])
    @pl.when(sid == NS - 1)
    def _():
      tb = pl.multiple_of(cid * N_NODES + NS * zch, 8)
      pltpu.sync_copy(hist_s.at[pl.ds(NS * zch, N_NODES - NS * zch)],
                      ones_v.at[pl.ds(0, N_NODES - NS * zch)])
      pltpu.sync_copy(ones_v.at[pl.ds(0, N_NODES - NS * zch)],
                      out_hbm.at[pl.ds(tb, N_NODES - NS * zch)])

  return hist_kernel(dst)


RING = 6
AHEAD = 4


def _sc_mp(g, src, dst):
  e = src.shape[0]
  ept = e // NS
  n_full = ept // CH
  tail = ept - n_full * CH
  rpt = 624

  assert n_full % RING == 0 and n_full >= 2 * RING
  outer = n_full // RING

  @functools.partial(
      pl.kernel,
      out_type=jax.ShapeDtypeStruct((NC, N_NODES, HF), jnp.float32),
      mesh=_MESH,
      compiler_params=pltpu.CompilerParams(use_tc_tiling_on_sc=False),
      scratch_types=[
          pltpu.VMEM((ept,), jnp.int32),
          [pltpu.VMEM((CH,), jnp.int32) for _ in range(RING)],
          [pltpu.VMEM((CH, HF), jnp.float32) for _ in range(RING)],
          pltpu.VMEM((tail,), jnp.int32),
          pltpu.VMEM_SHARED((N_NODES, HF), jnp.float32),
          [pltpu.SemaphoreType.DMA for _ in range(RING)],
          [pltpu.SemaphoreType.DMA for _ in range(RING)],
          [pltpu.SemaphoreType.DMA for _ in range(RING)],
      ],
  )
  def mp_kernel(g_hbm, src_hbm, dst_hbm, out_hbm,
                srcall, dstv, rows, tdst_v, acc_s, gsem, ssem, isem):
    cid = lax.axis_index("c")
    sid = lax.axis_index("s")
    ebase = pl.multiple_of(sid * ept, 8)
    rows0 = rows[0]
    ghalf = g_hbm.at[cid]

    _zero_rows(rows0, CH, HF)
    r0 = pl.multiple_of(sid * rpt, 8)
    for k in range(rpt // CH):
      pltpu.sync_copy(rows0, acc_s.at[pl.ds(r0 + k * CH, CH)])
    rem = rpt - (rpt // CH) * CH
    if rem:
      pltpu.sync_copy(rows0.at[pl.ds(0, rem)],
                      acc_s.at[pl.ds(r0 + (rpt // CH) * CH, rem)])
    ntail = N_NODES - NS * rpt
    @pl.when(sid == NS - 1)
    def _():
      pltpu.sync_copy(rows0.at[pl.ds(0, ntail)],
                      acc_s.at[pl.ds(NS * rpt, ntail)])

    pltpu.sync_copy(src_hbm.at[pl.ds(ebase, ept)], srcall)
    plsc.subcore_barrier()

    def g_start(c, k):
      pltpu.make_async_copy(
          ghalf.at[srcall.at[pl.ds(pl.multiple_of(c * CH, 8), CH)]],
          rows[k], gsem[k]).start()

    def g_wait(c, k):
      pltpu.make_async_copy(
          ghalf.at[srcall.at[pl.ds(pl.multiple_of(c * CH, 8), CH)]],
          rows[k], gsem[k]).wait()

    def d_start(c, k):
      b = pl.multiple_of(ebase + c * CH, 8)
      pltpu.make_async_copy(dst_hbm.at[pl.ds(b, CH)], dstv[k], isem[k]).start()

    def d_wait(c, k):
      b = pl.multiple_of(ebase + c * CH, 8)
      pltpu.make_async_copy(dst_hbm.at[pl.ds(b, CH)], dstv[k], isem[k]).wait()

    def s_start(k):
      pltpu.make_async_copy(rows[k], acc_s.at[dstv[k]], ssem[k]).start(add=True)

    def s_wait(k):
      pltpu.make_async_copy(rows[k], acc_s.at[dstv[k]], ssem[k]).wait()

    for c0 in range(AHEAD):
      d_start(c0, c0)
      g_start(c0, c0)

    def body(o, _):
      for k in range(RING):
        c = RING * o + k
        g_wait(c, k)
        d_wait(c, k)
        s_start(k)
        kn = (k + AHEAD) % RING
        if k < RING - AHEAD:
          @pl.when(o > 0)
          def _():
            s_wait(kn)
          d_start(c + AHEAD, kn)
          g_start(c + AHEAD, kn)
        else:
          s_wait(kn)
          @pl.when(o < outer - 1)
          def _():
            d_start(c + AHEAD, kn)
            g_start(c + AHEAD, kn)
      return 0
    lax.fori_loop(0, outer, body, 0)
    s_wait((n_full - 2) % RING)
    s_wait((n_full - 1) % RING)

    if tail:
      b = pl.multiple_of(ebase + n_full * CH, 8)
      pltpu.sync_copy(dst_hbm.at[pl.ds(b, tail)], tdst_v)
      pltpu.async_copy(
          ghalf.at[srcall.at[pl.ds(n_full * CH, tail)]],
          rows0.at[pl.ds(0, tail)], gsem[0]).wait()
      pltpu.sync_copy(rows0.at[pl.ds(0, tail)], acc_s.at[tdst_v], add=True)

    plsc.subcore_barrier()
    for k in range(rpt // CH):
      pltpu.sync_copy(acc_s.at[pl.ds(r0 + k * CH, CH)], rows0)
      pltpu.sync_copy(rows0, out_hbm.at[cid, pl.ds(r0 + k * CH, CH)])
    if rem:
      rb = pl.multiple_of(r0 + (rpt // CH) * CH, 8)
      pltpu.sync_copy(acc_s.at[pl.ds(rb, rem)], rows0.at[pl.ds(0, rem)])
      pltpu.sync_copy(rows0.at[pl.ds(0, rem)],
                      out_hbm.at[cid, pl.ds(rb, rem)])
    @pl.when(sid == NS - 1)
    def _():
      pltpu.sync_copy(acc_s.at[pl.ds(NS * rpt, ntail)],
                      rows0.at[pl.ds(0, ntail)])
      pltpu.sync_copy(rows0.at[pl.ds(0, ntail)],
                      out_hbm.at[cid, pl.ds(NS * rpt, ntail)])

  return mp_kernel(g, src, dst)


_R = 1000


def _tc_scale_matmul(hist_t, x, W1):
  def body(hist_ref, x_ref, w_ref, g_ref, dinv_ref):
    deg = hist_ref[:, 0:1] + hist_ref[:, 1:2] + 1.0
    dinv = lax.rsqrt(deg)
    dinv_ref[...] = dinv
    t = dinv * jnp.dot(x_ref[...], w_ref[...],
                       preferred_element_type=jnp.float32)
    g_ref[0] = t[:, :HF]
    g_ref[1] = t[:, HF:]

  return pl.pallas_call(
      body,
      grid=(N_NODES // _R,),
      in_specs=[
          pl.BlockSpec((_R, NC), lambda i: (i, 0)),
          pl.BlockSpec((_R, FEAT), lambda i: (i, 0)),
          pl.BlockSpec((FEAT, FEAT), lambda i: (0, 0)),
      ],
      out_specs=[
          pl.BlockSpec((NC, _R, HF), lambda i: (0, i, 0)),
          pl.BlockSpec((_R, 1), lambda i: (i, 0)),
      ],
      out_shape=[
          jax.ShapeDtypeStruct((NC, N_NODES, HF), jnp.float32),
          jax.ShapeDtypeStruct((N_NODES, 1), jnp.float32),
      ],
  )(hist_t, x, W1)


def _tc_layer1_matmul2(acc, g1s, dinv, b1, W2):
  def body(acc_ref, g_ref, dinv_ref, b_ref, w_ref, o_ref):
    m = jnp.concatenate(
        [acc_ref[0] + g_ref[0], acc_ref[1] + g_ref[1]], axis=-1)
    h = jnp.maximum(dinv_ref[...] * m + b_ref[...], 0.0)
    t = dinv_ref[...] * jnp.dot(h, w_ref[...],
                                preferred_element_type=jnp.float32)
    o_ref[0] = t[:, :HF]
    o_ref[1] = t[:, HF:]

  return pl.pallas_call(
      body,
      grid=(N_NODES // _R,),
      in_specs=[
          pl.BlockSpec((NC, _R, HF), lambda i: (0, i, 0)),
          pl.BlockSpec((NC, _R, HF), lambda i: (0, i, 0)),
          pl.BlockSpec((_R, 1), lambda i: (i, 0)),
          pl.BlockSpec((1, FEAT), lambda i: (0, 0)),
          pl.BlockSpec((FEAT, FEAT), lambda i: (0, 0)),
      ],
      out_specs=pl.BlockSpec((NC, _R, HF), lambda i: (0, i, 0)),
      out_shape=jax.ShapeDtypeStruct((NC, N_NODES, HF), jnp.float32),
  )(acc, g1s, dinv, b1, W2)


def _tc_layer2_logsoftmax(acc, g2s, dinv, b2):
  def body(acc_ref, g_ref, dinv_ref, b_ref, o_ref):
    m = jnp.concatenate(
        [acc_ref[0] + g_ref[0], acc_ref[1] + g_ref[1]], axis=-1)
    z = dinv_ref[...] * m + b_ref[...]
    zm = jnp.max(z, axis=-1, keepdims=True)
    ez = jnp.exp(z - zm)
    o_ref[...] = z - zm - jnp.log(jnp.sum(ez, axis=-1, keepdims=True))

  return pl.pallas_call(
      body,
      grid=(N_NODES // _R,),
      in_specs=[
          pl.BlockSpec((NC, _R, HF), lambda i: (0, i, 0)),
          pl.BlockSpec((NC, _R, HF), lambda i: (0, i, 0)),
          pl.BlockSpec((_R, 1), lambda i: (i, 0)),
          pl.BlockSpec((1, FEAT), lambda i: (0, 0)),
      ],
      out_specs=pl.BlockSpec((_R, FEAT), lambda i: (i, 0)),
      out_shape=jax.ShapeDtypeStruct((N_NODES, FEAT), jnp.float32),
  )(acc, g2s, dinv, b2)


def kernel(x, edge_index, W1, b1, W2, b2):
  ei = edge_index.astype(jnp.int32)
  src, dst = ei[0], ei[1]

  hist = _sc_hist(dst).reshape(NC, N_NODES)
  hist_t = hist.T
  g1s, dinv = _tc_scale_matmul(hist_t, x, W1)
  acc1 = _sc_mp(g1s, src, dst)
  g2s = _tc_layer1_matmul2(acc1, g1s, dinv, b1.reshape(1, FEAT), W2)
  acc2 = _sc_mp(g2s, src, dst)
  return _tc_layer2_logsoftmax(acc2, g2s, dinv, b2.reshape(1, FEAT))


# speedup vs baseline: 32.1180x; 1.1055x over previous
import functools

import jax
import jax.numpy as jnp
from jax import lax
from jax.experimental import pallas as pl
from jax.experimental.pallas import tpu as pltpu
from jax.experimental.pallas import tpu_sc as plsc

N_NODES = 10000
FEAT = 128
HF = FEAT // 2
NC = 2
NS = 16
NW = NC * NS
CH = 128

_MESH = plsc.VectorSubcoreMesh(
    core_axis_name="c", subcore_axis_name="s", num_cores=NC, num_subcores=NS)


def _zero_rows(buf, nrows, width):
  zero16 = jnp.zeros((16,), jnp.float32)

  def row(i, _):
    def col(j, _):
      buf[i, pl.ds(j * 16, 16)] = zero16
      return 0
    return lax.fori_loop(0, width // 16, col, 0)

  lax.fori_loop(0, nrows, row, 0)


def _sc_hist(dst):
  e = dst.shape[0]
  ept = e // NW
  n_full = ept // CH
  tail = ept - n_full * CH
  zch = 624

  assert n_full % RING == 0

  @functools.partial(
      pl.kernel,
      out_type=jax.ShapeDtypeStruct((NC * N_NODES,), jnp.float32),
      mesh=_MESH,
      scratch_types=[
          [pltpu.VMEM((CH,), jnp.int32) for _ in range(RING)],
          pltpu.VMEM((tail,), jnp.int32),
          pltpu.VMEM((CH,), jnp.float32),
          pltpu.VMEM((zch,), jnp.float32),
          pltpu.VMEM_SHARED((N_NODES,), jnp.float32),
          [pltpu.SemaphoreType.DMA for _ in range(RING)],
          [pltpu.SemaphoreType.DMA for _ in range(RING)],
      ],
  )
  def hist_kernel(dst_hbm, out_hbm, idxv, tidx_v, ones_v, zbuf, hist_s,
                  ssem, isem):
    cid = lax.axis_index("c")
    sid = lax.axis_index("s")
    wid = sid * NC + cid
    ebase = pl.multiple_of(wid * ept, 8)

    one16 = jnp.ones((16,), jnp.float32)
    zero16 = jnp.zeros((16,), jnp.float32)
    for j in range(CH // 16):
      ones_v[pl.ds(j * 16, 16)] = one16
    def zb(j, _):
      zbuf[pl.ds(j * 16, 16)] = zero16
      return 0
    lax.fori_loop(0, zch // 16, zb, 0)

    pltpu.sync_copy(zbuf, hist_s.at[pl.ds(sid * zch, zch)])
    @pl.when(sid == NS - 1)
    def _():
      pltpu.sync_copy(zbuf.at[pl.ds(0, 16)],
                      hist_s.at[pl.ds(NS * zch, N_NODES - NS * zch)])
    plsc.subcore_barrier()

    def d_start(c, k):
      b = pl.multiple_of(ebase + c * CH, 8)
      pltpu.make_async_copy(dst_hbm.at[pl.ds(b, CH)], idxv[k], isem[k]).start()

    def d_wait(c, k):
      b = pl.multiple_of(ebase + c * CH, 8)
      pltpu.make_async_copy(dst_hbm.at[pl.ds(b, CH)], idxv[k], isem[k]).wait()

    def s_start(k):
      pltpu.make_async_copy(
          ones_v, hist_s.at[idxv[k]], ssem[k]).start(add=True)

    def s_wait(k):
      pltpu.make_async_copy(ones_v, hist_s.at[idxv[k]], ssem[k]).wait()

    for c0 in range(AHEAD):
      d_start(c0, c0)

    def body(o, _):
      for k in range(RING):
        c = RING * o + k
        d_wait(c, k)
        s_start(k)
        kn = (k + AHEAD) % RING
        if k < RING - AHEAD:
          @pl.when(o > 0)
          def _():
            s_wait(kn)
          d_start(c + AHEAD, kn)
        else:
          s_wait(kn)
          @pl.when(o < n_full // RING - 1)
          def _():
            d_start(c + AHEAD, kn)
      return 0
    lax.fori_loop(0, n_full // RING, body, 0)
    s_wait((n_full - 2) % RING)
    s_wait((n_full - 1) % RING)

    if tail:
      base = pl.multiple_of(ebase + n_full * CH, 8)
      pltpu.sync_copy(dst_hbm.at[pl.ds(base, tail)], tidx_v)
      pltpu.sync_copy(ones_v.at[pl.ds(0, tail)], hist_s.at[tidx_v], add=True)

    plsc.subcore_barrier()
    obase = pl.multiple_of(cid * N_NODES + sid * zch, 8)
    pltpu.sync_copy(hist_s.at[pl.ds(sid * zch, zch)], zbuf)
    pltpu.sync_copy(zbuf, out_hbm.at[pl.ds(obase, zch)])
    @pl.when(sid == NS - 1)
    def _():
      tb = pl.multiple_of(cid * N_NODES + NS * zch, 8)
      pltpu.sync_copy(hist_s.at[pl.ds(NS * zch, N_NODES - NS * zch)],
                      ones_v.at[pl.ds(0, N_NODES - NS * zch)])
      pltpu.sync_copy(ones_v.at[pl.ds(0, N_NODES - NS * zch)],
                      out_hbm.at[pl.ds(tb, N_NODES - NS * zch)])

  return hist_kernel(dst)


RING = 6
AHEAD = 4


def _sc_mp(g, src, dst):
  e = src.shape[0]
  ept = e // NS
  n_full = ept // CH
  tail = ept - n_full * CH
  rpt = 624

  assert n_full % RING == 0 and n_full >= 2 * RING
  outer = n_full // RING

  @functools.partial(
      pl.kernel,
      out_type=jax.ShapeDtypeStruct((NC, N_NODES, HF), jnp.float32),
      mesh=_MESH,
      compiler_params=pltpu.CompilerParams(use_tc_tiling_on_sc=False),
      scratch_types=[
          pltpu.VMEM((ept,), jnp.int32),
          [pltpu.VMEM((CH,), jnp.int32) for _ in range(RING)],
          [pltpu.VMEM((CH, HF), jnp.float32) for _ in range(RING)],
          pltpu.VMEM((tail,), jnp.int32),
          pltpu.VMEM_SHARED((N_NODES, HF), jnp.float32),
          [pltpu.SemaphoreType.DMA for _ in range(RING)],
          [pltpu.SemaphoreType.DMA for _ in range(RING)],
          [pltpu.SemaphoreType.DMA for _ in range(RING)],
      ],
  )
  def mp_kernel(g_hbm, src_hbm, dst_hbm, out_hbm,
                srcall, dstv, rows, tdst_v, acc_s, gsem, ssem, isem):
    cid = lax.axis_index("c")
    sid = lax.axis_index("s")
    ebase = pl.multiple_of(sid * ept, 8)
    rows0 = rows[0]
    ghalf = g_hbm.at[cid]

    _zero_rows(rows0, CH, HF)
    r0 = pl.multiple_of(sid * rpt, 8)
    for k in range(rpt // CH):
      pltpu.sync_copy(rows0, acc_s.at[pl.ds(r0 + k * CH, CH)])
    rem = rpt - (rpt // CH) * CH
    if rem:
      pltpu.sync_copy(rows0.at[pl.ds(0, rem)],
                      acc_s.at[pl.ds(r0 + (rpt // CH) * CH, rem)])
    ntail = N_NODES - NS * rpt
    @pl.when(sid == NS - 1)
    def _():
      pltpu.sync_copy(rows0.at[pl.ds(0, ntail)],
                      acc_s.at[pl.ds(NS * rpt, ntail)])

    pltpu.sync_copy(src_hbm.at[pl.ds(ebase, ept)], srcall)
    plsc.subcore_barrier()

    def g_start(c, k):
      pltpu.make_async_copy(
          ghalf.at[srcall.at[pl.ds(pl.multiple_of(c * CH, 8), CH)]],
          rows[k], gsem[k]).start()

    def g_wait(c, k):
      pltpu.make_async_copy(
          ghalf.at[srcall.at[pl.ds(pl.multiple_of(c * CH, 8), CH)]],
          rows[k], gsem[k]).wait()

    def d_start(c, k):
      b = pl.multiple_of(ebase + c * CH, 8)
      pltpu.make_async_copy(dst_hbm.at[pl.ds(b, CH)], dstv[k], isem[k]).start()

    def d_wait(c, k):
      b = pl.multiple_of(ebase + c * CH, 8)
      pltpu.make_async_copy(dst_hbm.at[pl.ds(b, CH)], dstv[k], isem[k]).wait()

    def s_start(k):
      pltpu.make_async_copy(rows[k], acc_s.at[dstv[k]], ssem[k]).start(add=True)

    def s_wait(k):
      pltpu.make_async_copy(rows[k], acc_s.at[dstv[k]], ssem[k]).wait()

    for c0 in range(AHEAD):
      d_start(c0, c0)
      g_start(c0, c0)

    def body(o, _):
      for k in range(RING):
        c = RING * o + k
        g_wait(c, k)
        d_wait(c, k)
        s_start(k)
        kn = (k + AHEAD) % RING
        if k < RING - AHEAD:
          @pl.when(o > 0)
          def _():
            s_wait(kn)
          d_start(c + AHEAD, kn)
          g_start(c + AHEAD, kn)
        else:
          s_wait(kn)
          @pl.when(o < outer - 1)
          def _():
            d_start(c + AHEAD, kn)
            g_start(c + AHEAD, kn)
      return 0
    lax.fori_loop(0, outer, body, 0)
    s_wait((n_full - 2) % RING)
    s_wait((n_full - 1) % RING)

    if tail:
      b = pl.multiple_of(ebase + n_full * CH, 8)
      pltpu.sync_copy(dst_hbm.at[pl.ds(b, tail)], tdst_v)
      pltpu.async_copy(
          ghalf.at[srcall.at[pl.ds(n_full * CH, tail)]],
          rows0.at[pl.ds(0, tail)], gsem[0]).wait()
      pltpu.sync_copy(rows0.at[pl.ds(0, tail)], acc_s.at[tdst_v], add=True)

    plsc.subcore_barrier()
    for k in range(rpt // CH):
      pltpu.sync_copy(acc_s.at[pl.ds(r0 + k * CH, CH)], rows0)
      pltpu.sync_copy(rows0, out_hbm.at[cid, pl.ds(r0 + k * CH, CH)])
    if rem:
      rb = pl.multiple_of(r0 + (rpt // CH) * CH, 8)
      pltpu.sync_copy(acc_s.at[pl.ds(rb, rem)], rows0.at[pl.ds(0, rem)])
      pltpu.sync_copy(rows0.at[pl.ds(0, rem)],
                      out_hbm.at[cid, pl.ds(rb, rem)])
    @pl.when(sid == NS - 1)
    def _():
      pltpu.sync_copy(acc_s.at[pl.ds(NS * rpt, ntail)],
                      rows0.at[pl.ds(0, ntail)])
      pltpu.sync_copy(rows0.at[pl.ds(0, ntail)],
                      out_hbm.at[cid, pl.ds(NS * rpt, ntail)])

  return mp_kernel(g, src, dst)


_R = 1000


def _tc_scale_matmul(hist_t, x, W1):
  def body(hist_ref, x_ref, w_ref, g_ref, dinv_ref):
    deg = hist_ref[:, 0:1] + hist_ref[:, 1:2] + 1.0
    dinv = lax.rsqrt(deg)
    dinv_ref[...] = dinv
    t = dinv * jnp.dot(x_ref[...], w_ref[...],
                       preferred_element_type=jnp.float32)
    g_ref[0] = t[:, :HF]
    g_ref[1] = t[:, HF:]

  return pl.pallas_call(
      body,
      grid=(N_NODES // _R,),
      in_specs=[
          pl.BlockSpec((_R, NC), lambda i: (i, 0)),
          pl.BlockSpec((_R, FEAT), lambda i: (i, 0)),
          pl.BlockSpec((FEAT, FEAT), lambda i: (0, 0)),
      ],
      out_specs=[
          pl.BlockSpec((NC, _R, HF), lambda i: (0, i, 0)),
          pl.BlockSpec((_R, 1), lambda i: (i, 0)),
      ],
      out_shape=[
          jax.ShapeDtypeStruct((NC, N_NODES, HF), jnp.float32),
          jax.ShapeDtypeStruct((N_NODES, 1), jnp.float32),
      ],
  )(hist_t, x, W1)


def _tc_layer1_matmul2(acc, g1s, dinv, b1, W2):
  def body(acc_ref, g_ref, dinv_ref, b_ref, w_ref, o_ref):
    m = jnp.concatenate(
        [acc_ref[0] + g_ref[0], acc_ref[1] + g_ref[1]], axis=-1)
    h = jnp.maximum(dinv_ref[...] * m + b_ref[...], 0.0)
    t = dinv_ref[...] * jnp.dot(h, w_ref[...],
                                preferred_element_type=jnp.float32)
    o_ref[0] = t[:, :HF]
    o_ref[1] = t[:, HF:]

  return pl.pallas_call(
      body,
      grid=(N_NODES // _R,),
      in_specs=[
          pl.BlockSpec((NC, _R, HF), lambda i: (0, i, 0)),
          pl.BlockSpec((NC, _R, HF), lambda i: (0, i, 0)),
          pl.BlockSpec((_R, 1), lambda i: (i, 0)),
          pl.BlockSpec((1, FEAT), lambda i: (0, 0)),
          pl.BlockSpec((FEAT, FEAT), lambda i: (0, 0)),
      ],
      out_specs=pl.BlockSpec((NC, _R, HF), lambda i: (0, i, 0)),
      out_shape=jax.ShapeDtypeStruct((NC, N_NODES, HF), jnp.float32),
  )(acc, g1s, dinv, b1, W2)


def _tc_layer2_logsoftmax(acc, g2s, dinv, b2):
  def body(acc_ref, g_ref, dinv_ref, b_ref, o_ref):
    m = jnp.concatenate(
        [acc_ref[0] + g_ref[0], acc_ref[1] + g_ref[1]], axis=-1)
    z = dinv_ref[...] * m + b_ref[...]
    zm = jnp.max(z, axis=-1, keepdims=True)
    ez = jnp.exp(z - zm)
    o_ref[...] = z - zm - jnp.log(jnp.sum(ez, axis=-1, keepdims=True))

  return pl.pallas_call(
      body,
      grid=(N_NODES // _R,),
      in_specs=[
          pl.BlockSpec((NC, _R, HF), lambda i: (0, i, 0)),
          pl.BlockSpec((NC, _R, HF), lambda i: (0, i, 0)),
          pl.BlockSpec((_R, 1), lambda i: (i, 0)),
          pl.BlockSpec((1, FEAT), lambda i: (0, 0)),
      ],
      out_specs=pl.BlockSpec((_R, FEAT), lambda i: (i, 0)),
      out_shape=jax.ShapeDtypeStruct((N_NODES, FEAT), jnp.float32),
  )(acc, g2s, dinv, b2)


def kernel(x, edge_index, W1, b1, W2, b2):
  ei = edge_index.astype(jnp.int32)
  src, dst = ei[0], ei[1]

  hist = _sc_hist(dst).reshape(NC, N_NODES)
  hist_t = hist.T
  g1s, dinv = _tc_scale_matmul(hist_t, x, W1)
  acc1 = _sc_mp(g1s, src, dst)
  g2s = _tc_layer1_matmul2(acc1, g1s, dinv, b1.reshape(1, FEAT), W2)
  acc2 = _sc_mp(g2s, src, dst)
  return _tc_layer2_logsoftmax(acc2, g2s, dinv, b2.reshape(1, FEAT))


# speedup vs baseline: 32.9852x; 1.0270x over previous
import functools

import jax
import jax.numpy as jnp
from jax import lax
from jax.experimental import pallas as pl
from jax.experimental.pallas import tpu as pltpu
from jax.experimental.pallas import tpu_sc as plsc

N_NODES = 10000
FEAT = 128
HF = FEAT // 2
NC = 2
NS = 16
NW = NC * NS
CH = 128

_MESH = plsc.VectorSubcoreMesh(
    core_axis_name="c", subcore_axis_name="s", num_cores=NC, num_subcores=NS)


def _zero_rows(buf, nrows, width):
  zero16 = jnp.zeros((16,), jnp.float32)

  def row(i, _):
    def col(j, _):
      buf[i, pl.ds(j * 16, 16)] = zero16
      return 0
    return lax.fori_loop(0, width // 16, col, 0)

  lax.fori_loop(0, nrows, row, 0)


def _sc_hist(dst):
  e = dst.shape[0]
  ept = e // NW
  n_full = ept // CH
  tail = ept - n_full * CH
  zch = 624

  assert n_full % RING == 0

  @functools.partial(
      pl.kernel,
      out_type=jax.ShapeDtypeStruct((NC * N_NODES,), jnp.float32),
      mesh=_MESH,
      scratch_types=[
          [pltpu.VMEM((CH,), jnp.int32) for _ in range(RING)],
          pltpu.VMEM((tail,), jnp.int32),
          pltpu.VMEM((CH,), jnp.float32),
          pltpu.VMEM((zch,), jnp.float32),
          pltpu.VMEM_SHARED((N_NODES,), jnp.float32),
          [pltpu.SemaphoreType.DMA for _ in range(RING)],
          [pltpu.SemaphoreType.DMA for _ in range(RING)],
      ],
  )
  def hist_kernel(dst_hbm, out_hbm, idxv, tidx_v, ones_v, zbuf, hist_s,
                  ssem, isem):
    cid = lax.axis_index("c")
    sid = lax.axis_index("s")
    wid = sid * NC + cid
    ebase = pl.multiple_of(wid * ept, 8)

    one16 = jnp.ones((16,), jnp.float32)
    zero16 = jnp.zeros((16,), jnp.float32)
    for j in range(CH // 16):
      ones_v[pl.ds(j * 16, 16)] = one16
    def zb(j, _):
      zbuf[pl.ds(j * 16, 16)] = zero16
      return 0
    lax.fori_loop(0, zch // 16, zb, 0)

    pltpu.sync_copy(zbuf, hist_s.at[pl.ds(sid * zch, zch)])
    @pl.when(sid == NS - 1)
    def _():
      pltpu.sync_copy(zbuf.at[pl.ds(0, 16)],
                      hist_s.at[pl.ds(NS * zch, N_NODES - NS * zch)])
    plsc.subcore_barrier()

    def d_start(c, k):
      b = pl.multiple_of(ebase + c * CH, 8)
      pltpu.make_async_copy(dst_hbm.at[pl.ds(b, CH)], idxv[k], isem[k]).start()

    def d_wait(c, k):
      b = pl.multiple_of(ebase + c * CH, 8)
      pltpu.make_async_copy(dst_hbm.at[pl.ds(b, CH)], idxv[k], isem[k]).wait()

    def s_start(k):
      pltpu.make_async_copy(
          ones_v, hist_s.at[idxv[k]], ssem[k]).start(add=True)

    def s_wait(k):
      pltpu.make_async_copy(ones_v, hist_s.at[idxv[k]], ssem[k]).wait()

    for c0 in range(AHEAD):
      d_start(c0, c0)

    def body(o, _):
      for k in range(RING):
        c = RING * o + k
        d_wait(c, k)
        s_start(k)
        kn = (k + AHEAD) % RING
        if k < RING - AHEAD:
          @pl.when(o > 0)
          def _():
            s_wait(kn)
          d_start(c + AHEAD, kn)
        else:
          s_wait(kn)
          @pl.when(o < n_full // RING - 1)
          def _():
            d_start(c + AHEAD, kn)
      return 0
    lax.fori_loop(0, n_full // RING, body, 0)
    s_wait((n_full - 2) % RING)
    s_wait((n_full - 1) % RING)

    if tail:
      base = pl.multiple_of(ebase + n_full * CH, 8)
      pltpu.sync_copy(dst_hbm.at[pl.ds(base, tail)], tidx_v)
      pltpu.sync_copy(ones_v.at[pl.ds(0, tail)], hist_s.at[tidx_v], add=True)

    plsc.subcore_barrier()
    obase = pl.multiple_of(cid * N_NODES + sid * zch, 8)
    pltpu.sync_copy(hist_s.at[pl.ds(sid * zch, zch)], zbuf)
    pltpu.sync_copy(zbuf, out_hbm.at[pl.ds(obase, zch)])
    @pl.when(sid == NS - 1)
    def _():
      tb = pl.multiple_of(cid * N_NODES + NS * zch, 8)
      pltpu.sync_copy(hist_s.at[pl.ds(NS * zch, N_NODES - NS * zch)],
                      ones_v.at[pl.ds(0, N_NODES - NS * zch)])
      pltpu.sync_copy(ones_v.at[pl.ds(0, N_NODES - NS * zch)],
                      out_hbm.at[pl.ds(tb, N_NODES - NS * zch)])

  return hist_kernel(dst)


RING = 6
AHEAD = 4


def _sc_mp(g, src, dst):
  e = src.shape[0]
  ept = e // NS
  n_full = ept // CH
  tail = ept - n_full * CH
  rpt = 624

  assert n_full % RING == 0 and n_full >= 2 * RING
  outer = n_full // RING

  @functools.partial(
      pl.kernel,
      out_type=jax.ShapeDtypeStruct((NC, N_NODES, HF), jnp.float32),
      mesh=_MESH,
      compiler_params=pltpu.CompilerParams(use_tc_tiling_on_sc=False),
      scratch_types=[
          pltpu.VMEM((ept,), jnp.int32),
          [pltpu.VMEM((CH,), jnp.int32) for _ in range(RING)],
          [pltpu.VMEM((CH, HF), jnp.float32) for _ in range(RING)],
          pltpu.VMEM((tail,), jnp.int32),
          pltpu.VMEM_SHARED((N_NODES, HF), jnp.float32),
          [pltpu.SemaphoreType.DMA for _ in range(RING)],
          [pltpu.SemaphoreType.DMA for _ in range(RING)],
          [pltpu.SemaphoreType.DMA for _ in range(RING)],
      ],
  )
  def mp_kernel(g_hbm, src_hbm, dst_hbm, out_hbm,
                srcall, dstv, rows, tdst_v, acc_s, gsem, ssem, isem):
    cid = lax.axis_index("c")
    sid = lax.axis_index("s")
    ebase = pl.multiple_of(sid * ept, 8)
    rows0 = rows[0]
    ghalf = g_hbm.at[cid]

    _zero_rows(rows0, CH, HF)
    r0 = pl.multiple_of(sid * rpt, 8)
    for k in range(rpt // CH):
      pltpu.sync_copy(rows0, acc_s.at[pl.ds(r0 + k * CH, CH)])
    rem = rpt - (rpt // CH) * CH
    if rem:
      pltpu.sync_copy(rows0.at[pl.ds(0, rem)],
                      acc_s.at[pl.ds(r0 + (rpt // CH) * CH, rem)])
    ntail = N_NODES - NS * rpt
    @pl.when(sid == NS - 1)
    def _():
      pltpu.sync_copy(rows0.at[pl.ds(0, ntail)],
                      acc_s.at[pl.ds(NS * rpt, ntail)])

    pltpu.sync_copy(src_hbm.at[pl.ds(ebase, ept)], srcall)
    plsc.subcore_barrier()

    def g_start(c, k):
      pltpu.make_async_copy(
          ghalf.at[srcall.at[pl.ds(pl.multiple_of(c * CH, 8), CH)]],
          rows[k], gsem[k]).start()

    def g_wait(c, k):
      pltpu.make_async_copy(
          ghalf.at[srcall.at[pl.ds(pl.multiple_of(c * CH, 8), CH)]],
          rows[k], gsem[k]).wait()

    def d_start(c, k):
      b = pl.multiple_of(ebase + c * CH, 8)
      pltpu.make_async_copy(dst_hbm.at[pl.ds(b, CH)], dstv[k], isem[k]).start()

    def d_wait(c, k):
      b = pl.multiple_of(ebase + c * CH, 8)
      pltpu.make_async_copy(dst_hbm.at[pl.ds(b, CH)], dstv[k], isem[k]).wait()

    def s_start(k):
      pltpu.make_async_copy(rows[k], acc_s.at[dstv[k]], ssem[k]).start(add=True)

    def s_wait(k):
      pltpu.make_async_copy(rows[k], acc_s.at[dstv[k]], ssem[k]).wait()

    for c0 in range(AHEAD):
      d_start(c0, c0)
      g_start(c0, c0)

    def body(o, _):
      for k in range(RING):
        c = RING * o + k
        g_wait(c, k)
        d_wait(c, k)
        s_start(k)
        kn = (k + AHEAD) % RING
        if k < RING - AHEAD:
          @pl.when(o > 0)
          def _():
            s_wait(kn)
          d_start(c + AHEAD, kn)
          g_start(c + AHEAD, kn)
        else:
          s_wait(kn)
          @pl.when(o < outer - 1)
          def _():
            d_start(c + AHEAD, kn)
            g_start(c + AHEAD, kn)
      return 0
    lax.fori_loop(0, outer, body, 0)
    s_wait((n_full - 2) % RING)
    s_wait((n_full - 1) % RING)

    if tail:
      b = pl.multiple_of(ebase + n_full * CH, 8)
      pltpu.sync_copy(dst_hbm.at[pl.ds(b, tail)], tdst_v)
      pltpu.async_copy(
          ghalf.at[srcall.at[pl.ds(n_full * CH, tail)]],
          rows0.at[pl.ds(0, tail)], gsem[0]).wait()
      pltpu.sync_copy(rows0.at[pl.ds(0, tail)], acc_s.at[tdst_v], add=True)

    plsc.subcore_barrier()
    nw_full = rpt // CH
    rb = pl.multiple_of(r0 + nw_full * CH, 8)
    for k in range(nw_full):
      pltpu.make_async_copy(acc_s.at[pl.ds(r0 + k * CH, CH)], rows[k],
                            gsem[k]).start()
    if rem:
      pltpu.make_async_copy(acc_s.at[pl.ds(rb, rem)],
                            rows[nw_full].at[pl.ds(0, rem)],
                            gsem[nw_full]).start()
    for k in range(nw_full):
      pltpu.make_async_copy(acc_s.at[pl.ds(r0 + k * CH, CH)], rows[k],
                            gsem[k]).wait()
      pltpu.make_async_copy(rows[k], out_hbm.at[cid, pl.ds(r0 + k * CH, CH)],
                            ssem[k]).start()
    if rem:
      pltpu.make_async_copy(acc_s.at[pl.ds(rb, rem)],
                            rows[nw_full].at[pl.ds(0, rem)],
                            gsem[nw_full]).wait()
      pltpu.make_async_copy(rows[nw_full].at[pl.ds(0, rem)],
                            out_hbm.at[cid, pl.ds(rb, rem)],
                            ssem[nw_full]).start()
    @pl.when(sid == NS - 1)
    def _():
      pltpu.sync_copy(acc_s.at[pl.ds(NS * rpt, ntail)],
                      rows[RING - 1].at[pl.ds(0, ntail)])
      pltpu.sync_copy(rows[RING - 1].at[pl.ds(0, ntail)],
                      out_hbm.at[cid, pl.ds(NS * rpt, ntail)])
    for k in range(nw_full):
      pltpu.make_async_copy(rows[k], out_hbm.at[cid, pl.ds(r0 + k * CH, CH)],
                            ssem[k]).wait()
    if rem:
      pltpu.make_async_copy(rows[nw_full].at[pl.ds(0, rem)],
                            out_hbm.at[cid, pl.ds(rb, rem)],
                            ssem[nw_full]).wait()

  return mp_kernel(g, src, dst)


_R = 2000


def _tc_scale_matmul(hist_t, x, W1):
  def body(hist_ref, x_ref, w_ref, g_ref, dinv_ref):
    deg = hist_ref[:, 0:1] + hist_ref[:, 1:2] + 1.0
    dinv = lax.rsqrt(deg)
    dinv_ref[...] = dinv
    t = dinv * jnp.dot(x_ref[...], w_ref[...],
                       preferred_element_type=jnp.float32)
    g_ref[0] = t[:, :HF]
    g_ref[1] = t[:, HF:]

  return pl.pallas_call(
      body,
      grid=(N_NODES // _R,),
      in_specs=[
          pl.BlockSpec((_R, NC), lambda i: (i, 0)),
          pl.BlockSpec((_R, FEAT), lambda i: (i, 0)),
          pl.BlockSpec((FEAT, FEAT), lambda i: (0, 0)),
      ],
      out_specs=[
          pl.BlockSpec((NC, _R, HF), lambda i: (0, i, 0)),
          pl.BlockSpec((_R, 1), lambda i: (i, 0)),
      ],
      out_shape=[
          jax.ShapeDtypeStruct((NC, N_NODES, HF), jnp.float32),
          jax.ShapeDtypeStruct((N_NODES, 1), jnp.float32),
      ],
  )(hist_t, x, W1)


def _tc_layer1_matmul2(acc, g1s, dinv, b1, W2):
  def body(acc_ref, g_ref, dinv_ref, b_ref, w_ref, o_ref):
    m = jnp.concatenate(
        [acc_ref[0] + g_ref[0], acc_ref[1] + g_ref[1]], axis=-1)
    h = jnp.maximum(dinv_ref[...] * m + b_ref[...], 0.0)
    t = dinv_ref[...] * jnp.dot(h, w_ref[...],
                                preferred_element_type=jnp.float32)
    o_ref[0] = t[:, :HF]
    o_ref[1] = t[:, HF:]

  return pl.pallas_call(
      body,
      grid=(N_NODES // _R,),
      in_specs=[
          pl.BlockSpec((NC, _R, HF), lambda i: (0, i, 0)),
          pl.BlockSpec((NC, _R, HF), lambda i: (0, i, 0)),
          pl.BlockSpec((_R, 1), lambda i: (i, 0)),
          pl.BlockSpec((1, FEAT), lambda i: (0, 0)),
          pl.BlockSpec((FEAT, FEAT), lambda i: (0, 0)),
      ],
      out_specs=pl.BlockSpec((NC, _R, HF), lambda i: (0, i, 0)),
      out_shape=jax.ShapeDtypeStruct((NC, N_NODES, HF), jnp.float32),
  )(acc, g1s, dinv, b1, W2)


def _tc_layer2_logsoftmax(acc, g2s, dinv, b2):
  def body(acc_ref, g_ref, dinv_ref, b_ref, o_ref):
    m = jnp.concatenate(
        [acc_ref[0] + g_ref[0], acc_ref[1] + g_ref[1]], axis=-1)
    z = dinv_ref[...] * m + b_ref[...]
    zm = jnp.max(z, axis=-1, keepdims=True)
    ez = jnp.exp(z - zm)
    o_ref[...] = z - zm - jnp.log(jnp.sum(ez, axis=-1, keepdims=True))

  return pl.pallas_call(
      body,
      grid=(N_NODES // _R,),
      in_specs=[
          pl.BlockSpec((NC, _R, HF), lambda i: (0, i, 0)),
          pl.BlockSpec((NC, _R, HF), lambda i: (0, i, 0)),
          pl.BlockSpec((_R, 1), lambda i: (i, 0)),
          pl.BlockSpec((1, FEAT), lambda i: (0, 0)),
      ],
      out_specs=pl.BlockSpec((_R, FEAT), lambda i: (i, 0)),
      out_shape=jax.ShapeDtypeStruct((N_NODES, FEAT), jnp.float32),
  )(acc, g2s, dinv, b2)


def kernel(x, edge_index, W1, b1, W2, b2):
  ei = edge_index.astype(jnp.int32)
  src, dst = ei[0], ei[1]

  hist = _sc_hist(dst).reshape(NC, N_NODES)
  hist_t = hist.T
  g1s, dinv = _tc_scale_matmul(hist_t, x, W1)
  acc1 = _sc_mp(g1s, src, dst)
  g2s = _tc_layer1_matmul2(acc1, g1s, dinv, b1.reshape(1, FEAT), W2)
  acc2 = _sc_mp(g2s, src, dst)
  return _tc_layer2_logsoftmax(acc2, g2s, dinv, b2.reshape(1, FEAT))


# speedup vs baseline: 32.9932x; 1.0002x over previous
import functools

import jax
import jax.numpy as jnp
from jax import lax
from jax.experimental import pallas as pl
from jax.experimental.pallas import tpu as pltpu
from jax.experimental.pallas import tpu_sc as plsc

N_NODES = 10000
FEAT = 128
HF = FEAT // 2
NC = 2
NS = 16
NW = NC * NS
CH = 128
RING = 6
AHEAD = 4

_MESH = plsc.VectorSubcoreMesh(
    core_axis_name="c", subcore_axis_name="s", num_cores=NC, num_subcores=NS)


def _zero_rows(buf, nrows, width):
  zero16 = jnp.zeros((16,), jnp.float32)

  def row(i, _):
    def col(j, _):
      buf[i, pl.ds(j * 16, 16)] = zero16
      return 0
    return lax.fori_loop(0, width // 16, col, 0)

  lax.fori_loop(0, nrows, row, 0)


def _sc_hist(dst):
  e = dst.shape[0]
  ept = e // NW
  n_full = ept // CH
  tail = ept - n_full * CH
  zch = 624

  assert n_full % RING == 0

  @functools.partial(
      pl.kernel,
      out_type=jax.ShapeDtypeStruct((NC * N_NODES,), jnp.float32),
      mesh=_MESH,
      scratch_types=[
          [pltpu.VMEM((CH,), jnp.int32) for _ in range(RING)],
          pltpu.VMEM((tail,), jnp.int32),
          pltpu.VMEM((CH,), jnp.float32),
          pltpu.VMEM((zch,), jnp.float32),
          pltpu.VMEM_SHARED((N_NODES,), jnp.float32),
          [pltpu.SemaphoreType.DMA for _ in range(RING)],
          [pltpu.SemaphoreType.DMA for _ in range(RING)],
      ],
  )
  def hist_kernel(dst_hbm, out_hbm, idxv, tidx_v, ones_v, zbuf, hist_s,
                  ssem, isem):
    cid = lax.axis_index("c")
    sid = lax.axis_index("s")
    wid = sid * NC + cid
    ebase = pl.multiple_of(wid * ept, 8)

    one16 = jnp.ones((16,), jnp.float32)
    zero16 = jnp.zeros((16,), jnp.float32)
    for j in range(CH // 16):
      ones_v[pl.ds(j * 16, 16)] = one16
    def zb(j, _):
      zbuf[pl.ds(j * 16, 16)] = zero16
      return 0
    lax.fori_loop(0, zch // 16, zb, 0)

    pltpu.sync_copy(zbuf, hist_s.at[pl.ds(sid * zch, zch)])
    @pl.when(sid == NS - 1)
    def _():
      pltpu.sync_copy(zbuf.at[pl.ds(0, 16)],
                      hist_s.at[pl.ds(NS * zch, N_NODES - NS * zch)])
    plsc.subcore_barrier()

    def d_start(c, k):
      b = pl.multiple_of(ebase + c * CH, 8)
      pltpu.make_async_copy(dst_hbm.at[pl.ds(b, CH)], idxv[k], isem[k]).start()

    def d_wait(c, k):
      b = pl.multiple_of(ebase + c * CH, 8)
      pltpu.make_async_copy(dst_hbm.at[pl.ds(b, CH)], idxv[k], isem[k]).wait()

    def s_start(k):
      pltpu.make_async_copy(
          ones_v, hist_s.at[idxv[k]], ssem[k]).start(add=True)

    def s_wait(k):
      pltpu.make_async_copy(ones_v, hist_s.at[idxv[k]], ssem[k]).wait()

    for c0 in range(AHEAD):
      d_start(c0, c0)

    def body(o, _):
      for k in range(RING):
        c = RING * o + k
        d_wait(c, k)
        s_start(k)
        kn = (k + AHEAD) % RING
        if k < RING - AHEAD:
          @pl.when(o > 0)
          def _():
            s_wait(kn)
          d_start(c + AHEAD, kn)
        else:
          s_wait(kn)
          @pl.when(o < n_full // RING - 1)
          def _():
            d_start(c + AHEAD, kn)
      return 0
    lax.fori_loop(0, n_full // RING, body, 0)
    s_wait((n_full - 2) % RING)
    s_wait((n_full - 1) % RING)

    if tail:
      base = pl.multiple_of(ebase + n_full * CH, 8)
      pltpu.sync_copy(dst_hbm.at[pl.ds(base, tail)], tidx_v)
      pltpu.sync_copy(ones_v.at[pl.ds(0, tail)], hist_s.at[tidx_v], add=True)

    plsc.subcore_barrier()
    obase = pl.multiple_of(cid * N_NODES + sid * zch, 8)
    pltpu.sync_copy(hist_s.at[pl.ds(sid * zch, zch)], zbuf)
    pltpu.sync_copy(zbuf, out_hbm.at[pl.ds(obase, zch)])
    @pl.when(sid == NS - 1)
    def _():
      tb = pl.multiple_of(cid * N_NODES + NS * zch, 8)
      pltpu.sync_copy(hist_s.at[pl.ds(NS * zch, N_NODES - NS * zch)],
                      ones_v.at[pl.ds(0, N_NODES - NS * zch)])
      pltpu.sync_copy(ones_v.at[pl.ds(0, N_NODES - NS * zch)],
                      out_hbm.at[pl.ds(tb, N_NODES - NS * zch)])

  return hist_kernel(dst)


def _sc_mp(g, src, dst):
  e = src.shape[0]
  ept = e // NS
  n_full = ept // CH
  tail = ept - n_full * CH
  rpt = 624

  assert n_full % RING == 0 and n_full >= 2 * RING
  outer = n_full // RING

  @functools.partial(
      pl.kernel,
      out_type=jax.ShapeDtypeStruct((NC, N_NODES, HF), jnp.float32),
      mesh=_MESH,
      compiler_params=pltpu.CompilerParams(use_tc_tiling_on_sc=False),
      scratch_types=[
          pltpu.VMEM((ept,), jnp.int32),
          [pltpu.VMEM((CH,), jnp.int32) for _ in range(RING)],
          [pltpu.VMEM((CH, HF), jnp.float32) for _ in range(RING)],
          pltpu.VMEM((tail,), jnp.int32),
          pltpu.VMEM_SHARED((N_NODES, HF), jnp.float32),
          [pltpu.SemaphoreType.DMA for _ in range(RING)],
          [pltpu.SemaphoreType.DMA for _ in range(RING)],
          [pltpu.SemaphoreType.DMA for _ in range(RING)],
      ],
  )
  def mp_kernel(g_hbm, src_hbm, dst_hbm, out_hbm,
                srcall, dstv, rows, tdst_v, acc_s, gsem, ssem, isem):
    cid = lax.axis_index("c")
    sid = lax.axis_index("s")
    ebase = pl.multiple_of(sid * ept, 8)
    rows0 = rows[0]

    _zero_rows(rows0, CH, HF)
    r0 = pl.multiple_of(sid * rpt, 8)
    for k in range(rpt // CH):
      pltpu.sync_copy(rows0, acc_s.at[pl.ds(r0 + k * CH, CH)])
    rem = rpt - (rpt // CH) * CH
    if rem:
      pltpu.sync_copy(rows0.at[pl.ds(0, rem)],
                      acc_s.at[pl.ds(r0 + (rpt // CH) * CH, rem)])
    ntail = N_NODES - NS * rpt
    @pl.when(sid == NS - 1)
    def _():
      pltpu.sync_copy(rows0.at[pl.ds(0, ntail)],
                      acc_s.at[pl.ds(NS * rpt, ntail)])

    pltpu.sync_copy(src_hbm.at[pl.ds(ebase, ept)], srcall)
    plsc.subcore_barrier()

    def g_start(c, k):
      pltpu.make_async_copy(
          g_hbm.at[cid].at[srcall.at[pl.ds(pl.multiple_of(c * CH, 8), CH)]],
          rows[k], gsem[k]).start()

    def g_wait(c, k):
      pltpu.make_async_copy(
          g_hbm.at[cid].at[srcall.at[pl.ds(pl.multiple_of(c * CH, 8), CH)]],
          rows[k], gsem[k]).wait()

    def d_start(c, k):
      b = pl.multiple_of(ebase + c * CH, 8)
      pltpu.make_async_copy(dst_hbm.at[pl.ds(b, CH)], dstv[k], isem[k]).start()

    def d_wait(c, k):
      b = pl.multiple_of(ebase + c * CH, 8)
      pltpu.make_async_copy(dst_hbm.at[pl.ds(b, CH)], dstv[k], isem[k]).wait()

    def s_start(k):
      pltpu.make_async_copy(rows[k], acc_s.at[dstv[k]], ssem[k]).start(add=True)

    def s_wait(k):
      pltpu.make_async_copy(rows[k], acc_s.at[dstv[k]], ssem[k]).wait()

    for c0 in range(AHEAD):
      d_start(c0, c0)
      g_start(c0, c0)

    def body(o, _):
      for k in range(RING):
        c = RING * o + k
        g_wait(c, k)
        d_wait(c, k)
        s_start(k)
        kn = (k + AHEAD) % RING
        if k < RING - AHEAD:
          @pl.when(o > 0)
          def _():
            s_wait(kn)
          d_start(c + AHEAD, kn)
          g_start(c + AHEAD, kn)
        else:
          s_wait(kn)
          @pl.when(o < outer - 1)
          def _():
            d_start(c + AHEAD, kn)
            g_start(c + AHEAD, kn)
      return 0
    lax.fori_loop(0, outer, body, 0)
    s_wait((n_full - 2) % RING)
    s_wait((n_full - 1) % RING)

    if tail:
      b = pl.multiple_of(ebase + n_full * CH, 8)
      pltpu.sync_copy(dst_hbm.at[pl.ds(b, tail)], tdst_v)
      pltpu.async_copy(
          g_hbm.at[cid].at[srcall.at[pl.ds(n_full * CH, tail)]],
          rows0.at[pl.ds(0, tail)], gsem[0]).wait()
      pltpu.sync_copy(rows0.at[pl.ds(0, tail)], acc_s.at[tdst_v], add=True)

    plsc.subcore_barrier()
    nw_full = rpt // CH
    rb = pl.multiple_of(r0 + nw_full * CH, 8)
    for k in range(nw_full):
      pltpu.make_async_copy(acc_s.at[pl.ds(r0 + k * CH, CH)], rows[k],
                            gsem[k]).start()
    if rem:
      pltpu.make_async_copy(acc_s.at[pl.ds(rb, rem)],
                            rows[nw_full].at[pl.ds(0, rem)],
                            gsem[nw_full]).start()
    for k in range(nw_full):
      pltpu.make_async_copy(acc_s.at[pl.ds(r0 + k * CH, CH)], rows[k],
                            gsem[k]).wait()
      pltpu.make_async_copy(rows[k], out_hbm.at[cid, pl.ds(r0 + k * CH, CH)],
                            ssem[k]).start()
    if rem:
      pltpu.make_async_copy(acc_s.at[pl.ds(rb, rem)],
                            rows[nw_full].at[pl.ds(0, rem)],
                            gsem[nw_full]).wait()
      pltpu.make_async_copy(rows[nw_full].at[pl.ds(0, rem)],
                            out_hbm.at[cid, pl.ds(rb, rem)],
                            ssem[nw_full]).start()
    @pl.when(sid == NS - 1)
    def _():
      pltpu.sync_copy(acc_s.at[pl.ds(NS * rpt, ntail)],
                      rows[RING - 1].at[pl.ds(0, ntail)])
      pltpu.sync_copy(rows[RING - 1].at[pl.ds(0, ntail)],
                      out_hbm.at[cid, pl.ds(NS * rpt, ntail)])
    for k in range(nw_full):
      pltpu.make_async_copy(rows[k], out_hbm.at[cid, pl.ds(r0 + k * CH, CH)],
                            ssem[k]).wait()
    if rem:
      pltpu.make_async_copy(rows[nw_full].at[pl.ds(0, rem)],
                            out_hbm.at[cid, pl.ds(rb, rem)],
                            ssem[nw_full]).wait()

  return mp_kernel(g, src, dst)


_R = 2000


def _tc_scale_matmul(hist_t, x, W1):
  def body(hist_ref, x_ref, w_ref, g_ref, dinv_ref):
    deg = hist_ref[:, 0:1] + hist_ref[:, 1:2] + 1.0
    dinv = lax.rsqrt(deg)
    dinv_ref[...] = dinv
    t = dinv * jnp.dot(x_ref[...], w_ref[...],
                       preferred_element_type=jnp.float32)
    g_ref[0] = t[:, :HF]
    g_ref[1] = t[:, HF:]

  return pl.pallas_call(
      body,
      grid=(N_NODES // _R,),
      in_specs=[
          pl.BlockSpec((_R, NC), lambda i: (i, 0)),
          pl.BlockSpec((_R, FEAT), lambda i: (i, 0)),
          pl.BlockSpec((FEAT, FEAT), lambda i: (0, 0)),
      ],
      out_specs=[
          pl.BlockSpec((NC, _R, HF), lambda i: (0, i, 0)),
          pl.BlockSpec((_R, 1), lambda i: (i, 0)),
      ],
      out_shape=[
          jax.ShapeDtypeStruct((NC, N_NODES, HF), jnp.float32),
          jax.ShapeDtypeStruct((N_NODES, 1), jnp.float32),
      ],
  )(hist_t, x, W1)


def _tc_layer1_matmul2(acc, g1s, dinv, b1, W2):
  def body(acc_ref, g_ref, dinv_ref, b_ref, w_ref, o_ref):
    m = jnp.concatenate(
        [acc_ref[0] + g_ref[0], acc_ref[1] + g_ref[1]], axis=-1)
    h = jnp.maximum(dinv_ref[...] * m + b_ref[...], 0.0)
    t = dinv_ref[...] * jnp.dot(h, w_ref[...],
                                preferred_element_type=jnp.float32)
    o_ref[0] = t[:, :HF]
    o_ref[1] = t[:, HF:]

  return pl.pallas_call(
      body,
      grid=(N_NODES // _R,),
      in_specs=[
          pl.BlockSpec((NC, _R, HF), lambda i: (0, i, 0)),
          pl.BlockSpec((NC, _R, HF), lambda i: (0, i, 0)),
          pl.BlockSpec((_R, 1), lambda i: (i, 0)),
          pl.BlockSpec((1, FEAT), lambda i: (0, 0)),
          pl.BlockSpec((FEAT, FEAT), lambda i: (0, 0)),
      ],
      out_specs=pl.BlockSpec((NC, _R, HF), lambda i: (0, i, 0)),
      out_shape=jax.ShapeDtypeStruct((NC, N_NODES, HF), jnp.float32),
  )(acc, g1s, dinv, b1, W2)


def _tc_layer2_logsoftmax(acc, g2s, dinv, b2):
  def body(acc_ref, g_ref, dinv_ref, b_ref, o_ref):
    m = jnp.concatenate(
        [acc_ref[0] + g_ref[0], acc_ref[1] + g_ref[1]], axis=-1)
    z = dinv_ref[...] * m + b_ref[...]
    zm = jnp.max(z, axis=-1, keepdims=True)
    ez = jnp.exp(z - zm)
    o_ref[...] = z - zm - jnp.log(jnp.sum(ez, axis=-1, keepdims=True))

  return pl.pallas_call(
      body,
      grid=(N_NODES // _R,),
      in_specs=[
          pl.BlockSpec((NC, _R, HF), lambda i: (0, i, 0)),
          pl.BlockSpec((NC, _R, HF), lambda i: (0, i, 0)),
          pl.BlockSpec((_R, 1), lambda i: (i, 0)),
          pl.BlockSpec((1, FEAT), lambda i: (0, 0)),
      ],
      out_specs=pl.BlockSpec((_R, FEAT), lambda i: (i, 0)),
      out_shape=jax.ShapeDtypeStruct((N_NODES, FEAT), jnp.float32),
  )(acc, g2s, dinv, b2)


def kernel(x, edge_index, W1, b1, W2, b2):
  ei = edge_index.astype(jnp.int32)
  src, dst = ei[0], ei[1]

  hist = _sc_hist(dst).reshape(NC, N_NODES)
  hist_t = hist.T
  g1s, dinv = _tc_scale_matmul(hist_t, x, W1)
  acc1 = _sc_mp(g1s, src, dst)
  g2s = _tc_layer1_matmul2(acc1, g1s, dinv, b1.reshape(1, FEAT), W2)
  acc2 = _sc_mp(g2s, src, dst)
  return _tc_layer2_logsoftmax(acc2, g2s, dinv, b2.reshape(1, FEAT))
